# baseline (speedup 1.0000x reference)
import jax
import jax.numpy as jnp
from jax.experimental import pallas as pl


def _cost_body(boxes_ref, tgt_ref, out_ref):
    b = boxes_ref[...]
    t = tgt_ref[...]
    bcx = (b[:, 0:1] + b[:, 2:3]) * 0.5
    bcy = (b[:, 1:2] + b[:, 3:4]) * 0.5
    bw = b[:, 2:3] - b[:, 0:1]
    bh = b[:, 3:4] - b[:, 1:2]
    tcx = (t[0:1, :] + t[2:3, :]) * 0.5
    tcy = (t[1:2, :] + t[3:4, :]) * 0.5
    tw = t[2:3, :] - t[0:1, :]
    th = t[3:4, :] - t[1:2, :]
    out_ref[...] = (
        jnp.abs(bcx - tcx)
        + jnp.abs(bcy - tcy)
        + jnp.abs(bw - tw)
        + jnp.abs(bh - th)
    )


def kernel(pre_boxes, anchors, targets):
    bs, num_queries = pre_boxes.shape[:2]
    m = targets.shape[0]
    n = bs * num_queries
    boxes = jnp.concatenate(
        [pre_boxes.reshape(n, 4), anchors.reshape(n, 4)], axis=0
    )
    tgt_t = targets.T

    block_n = 2000
    total = 2 * n
    grid = (total // block_n,)
    out = pl.pallas_call(
        _cost_body,
        grid=grid,
        in_specs=[
            pl.BlockSpec((block_n, 4), lambda i: (i, 0)),
            pl.BlockSpec((4, m), lambda i: (0, 0)),
        ],
        out_specs=pl.BlockSpec((block_n, m), lambda i: (i, 0)),
        out_shape=jax.ShapeDtypeStruct((total, m), jnp.float32),
    )(boxes, tgt_t)
    return out.reshape(2, bs, num_queries, m)


# speedup vs baseline: 2.5271x; 2.5271x over previous
import jax
import jax.numpy as jnp
from jax.experimental import pallas as pl


def _cost_body(pre_ref, anch_ref, tgt_ref, out_ref):
    i = pl.program_id(0)
    b = jnp.where(i < 8, pre_ref[0], anch_ref[0])
    t = tgt_ref[...]
    bcx = (b[:, 0:1] + b[:, 2:3]) * 0.5
    bcy = (b[:, 1:2] + b[:, 3:4]) * 0.5
    bw = b[:, 2:3] - b[:, 0:1]
    bh = b[:, 3:4] - b[:, 1:2]
    tcx = (t[0:1, :] + t[2:3, :]) * 0.5
    tcy = (t[1:2, :] + t[3:4, :]) * 0.5
    tw = t[2:3, :] - t[0:1, :]
    th = t[3:4, :] - t[1:2, :]
    out_ref[0, 0] = (
        jnp.abs(bcx - tcx)
        + jnp.abs(bcy - tcy)
        + jnp.abs(bw - tw)
        + jnp.abs(bh - th)
    )


def kernel(pre_boxes, anchors, targets):
    bs, num_queries = pre_boxes.shape[:2]
    m = targets.shape[0]
    tgt_t = targets.T

    out = pl.pallas_call(
        _cost_body,
        grid=(2 * bs,),
        in_specs=[
            pl.BlockSpec((1, num_queries, 4), lambda i: (i % bs, 0, 0)),
            pl.BlockSpec((1, num_queries, 4), lambda i: (i % bs, 0, 0)),
            pl.BlockSpec((4, m), lambda i: (0, 0)),
        ],
        out_specs=pl.BlockSpec(
            (1, 1, num_queries, m), lambda i: (i // bs, i % bs, 0, 0)
        ),
        out_shape=jax.ShapeDtypeStruct((2, bs, num_queries, m), jnp.float32),
    )(pre_boxes, anchors, tgt_t)
    return out


# speedup vs baseline: 2.6038x; 1.0304x over previous
import jax
import jax.numpy as jnp
from jax.experimental import pallas as pl


def _l1(b, t):
    bcx = (b[:, 0:1] + b[:, 2:3]) * 0.5
    bcy = (b[:, 1:2] + b[:, 3:4]) * 0.5
    bw = b[:, 2:3] - b[:, 0:1]
    bh = b[:, 3:4] - b[:, 1:2]
    tcx = (t[0:1, :] + t[2:3, :]) * 0.5
    tcy = (t[1:2, :] + t[3:4, :]) * 0.5
    tw = t[2:3, :] - t[0:1, :]
    th = t[3:4, :] - t[1:2, :]
    return (
        jnp.abs(bcx - tcx)
        + jnp.abs(bcy - tcy)
        + jnp.abs(bw - tw)
        + jnp.abs(bh - th)
    )


def _cost_body(pre_ref, anch_ref, tgt_ref, out_ref):
    t = tgt_ref[...]
    out_ref[0, 0] = _l1(pre_ref[0], t)
    out_ref[1, 0] = _l1(anch_ref[0], t)


def kernel(pre_boxes, anchors, targets):
    bs, num_queries = pre_boxes.shape[:2]
    m = targets.shape[0]
    tgt_t = targets.T

    out = pl.pallas_call(
        _cost_body,
        grid=(bs,),
        in_specs=[
            pl.BlockSpec((1, num_queries, 4), lambda i: (i, 0, 0)),
            pl.BlockSpec((1, num_queries, 4), lambda i: (i, 0, 0)),
            pl.BlockSpec((4, m), lambda i: (0, 0)),
        ],
        out_specs=pl.BlockSpec(
            (2, 1, num_queries, m), lambda i: (0, i, 0, 0)
        ),
        out_shape=jax.ShapeDtypeStruct((2, bs, num_queries, m), jnp.float32),
    )(pre_boxes, anchors, tgt_t)
    return out
